# Initial kernel scaffold; baseline (speedup 1.0000x reference)
import jax
import jax.numpy as jnp
from jax.experimental import pallas as pl


def kernel(states, signals, emb1, emb2, emb3):
    raise NotImplementedError("write your pallas kernel here")


# speedup vs baseline: 3.9597x; 3.9597x over previous
import functools

import jax
import jax.numpy as jnp
from jax import lax
from jax.experimental import pallas as pl
from jax.experimental.pallas import tpu as pltpu
from jax.experimental.pallas import tpu_sc as plsc

S = 4
D = 128
NSIG = 3
NLANE = 16
NCOL = D // NLANE

NC = 2
NS = 16
NW = NC * NS

C = 32
G = C * NSIG


def _build_kernel(n_pos):
    per_w = n_pos // NW
    n_chunks = per_w // C
    mesh = plsc.VectorSubcoreMesh(core_axis_name="c", subcore_axis_name="s")
    f32 = jnp.float32

    @functools.partial(
        pl.kernel,
        mesh=mesh,
        out_type=[jax.ShapeDtypeStruct((n_pos, D), f32)] * 3,
        scratch_types=[
            pltpu.VMEM((C, D), f32),
            pltpu.VMEM((C, D), f32),
            pltpu.VMEM((C, D), f32),
            pltpu.VMEM((C, D), f32),
            pltpu.VMEM((G, D), f32),
            pltpu.VMEM((G, D), f32),
            pltpu.VMEM((G, D), f32),
            pltpu.VMEM((G,), jnp.int32),
            pltpu.VMEM((G,), jnp.int32),
            pltpu.VMEM((G,), jnp.int32),
            pltpu.SemaphoreType.DMA,
        ],
    )
    def sc_kernel(states_h, idx1_h, idx2_h, idx3_h, e1_h, e2_h, e3_h,
                  o1_h, o2_h, o3_h,
                  st_v, o1_v, o2_v, o3_v, r1_v, r2_v, r3_v,
                  i1_v, i2_v, i3_v, sem):
        wid = lax.axis_index("s") * NC + lax.axis_index("c")
        w_base = wid * per_w

        def chunk_body(t, _):
            base = w_base + t * C
            pltpu.sync_copy(states_h.at[pl.ds(base, C)], st_v)
            pltpu.sync_copy(idx1_h.at[pl.ds(base * NSIG, G)], i1_v)
            pltpu.sync_copy(idx2_h.at[pl.ds(base * NSIG, G)], i2_v)
            pltpu.sync_copy(idx3_h.at[pl.ds(base * NSIG, G)], i3_v)
            pltpu.async_copy(e1_h.at[i1_v], r1_v, sem)
            pltpu.async_copy(e2_h.at[i2_v], r2_v, sem)
            cp3 = pltpu.async_copy(e3_h.at[i3_v], r3_v, sem)
            cp3.wait()
            cp3.wait()
            cp3.wait()

            def pos_body(j, _):
                r = j * NSIG
                for c in range(NCOL):
                    sl = pl.ds(c * NLANE, NLANE)
                    acc = st_v[j, sl]
                    acc = acc + r1_v[r, sl] + r1_v[r + 1, sl] + r1_v[r + 2, sl]
                    o1_v[j, sl] = acc
                    acc = acc + r2_v[r, sl] + r2_v[r + 1, sl] + r2_v[r + 2, sl]
                    o2_v[j, sl] = acc
                    acc = acc + r3_v[r, sl] + r3_v[r + 1, sl] + r3_v[r + 2, sl]
                    o3_v[j, sl] = acc
                return 0

            lax.fori_loop(0, C, pos_body, 0)
            pltpu.sync_copy(o1_v, o1_h.at[pl.ds(base, C)])
            pltpu.sync_copy(o2_v, o2_h.at[pl.ds(base, C)])
            pltpu.sync_copy(o3_v, o3_h.at[pl.ds(base, C)])
            return 0

        lax.fori_loop(0, n_chunks, chunk_body, 0)

    return sc_kernel


def kernel(states, signals, emb1, emb2, emb3):
    b, t, d = states.shape
    n_pos = b * t
    st_flat = states.reshape(n_pos, d)
    sig = signals.astype(jnp.int32)
    idx1 = sig[:, 1::S, :].reshape(-1)
    idx2 = sig[:, 2::S, :].reshape(-1)
    idx3 = sig[:, 3::S, :].reshape(-1)
    o1, o2, o3 = _build_kernel(n_pos)(
        st_flat, idx1, idx2, idx3, emb1, emb2, emb3)
    shape = states.shape
    return (states, o1.reshape(shape), o2.reshape(shape), o3.reshape(shape))


# speedup vs baseline: 5.7859x; 1.4612x over previous
import functools

import jax
import jax.numpy as jnp
from jax import lax
from jax.experimental import pallas as pl
from jax.experimental.pallas import tpu as pltpu
from jax.experimental.pallas import tpu_sc as plsc

S = 4
D = 128
NSIG = 3
NLANE = 16
NCOL = D // NLANE

NC = 2
NS = 16
NW = NC * NS

C = 16
G = C * NSIG
LEVELS = 256


def _build_kernel(n_pos):
    per_w = n_pos // NW
    n_chunks = per_w // C
    assert n_chunks % 2 == 0
    mesh = plsc.VectorSubcoreMesh(core_axis_name="c", subcore_axis_name="s")
    f32 = jnp.float32
    i32 = jnp.int32

    @functools.partial(
        pl.kernel,
        mesh=mesh,
        out_type=[jax.ShapeDtypeStruct((n_pos, D), f32)] * 3,
        scratch_types=[
            pltpu.VMEM((LEVELS, D), f32),
            pltpu.VMEM((LEVELS, D), f32),
            pltpu.VMEM((LEVELS, D), f32),
            pltpu.VMEM((2, C, D), f32),
            pltpu.VMEM((2, G + NLANE), i32),
            pltpu.VMEM((2, G + NLANE), i32),
            pltpu.VMEM((2, G + NLANE), i32),
            pltpu.VMEM((2, C, D), f32),
            pltpu.VMEM((2, C, D), f32),
            pltpu.VMEM((2, C, D), f32),
            pltpu.SemaphoreType.DMA,
            pltpu.SemaphoreType.DMA,
            pltpu.SemaphoreType.DMA,
        ],
    )
    def sc_kernel(states_h, idx1_h, idx2_h, idx3_h, e1_h, e2_h, e3_h,
                  o1_h, o2_h, o3_h,
                  e1_v, e2_v, e3_v, st_v, i1_v, i2_v, i3_v,
                  o1_v, o2_v, o3_v, sem_in, sem_o0, sem_o1):
        wid = lax.axis_index("s") * NC + lax.axis_index("c")
        w_base = wid * per_w
        sem_out = (sem_o0, sem_o1)

        pltpu.sync_copy(e1_h, e1_v)
        pltpu.sync_copy(e2_h, e2_v)
        pltpu.sync_copy(e3_h, e3_v)

        def fire_in(t, p):
            base = w_base + t * C
            pltpu.async_copy(states_h.at[pl.ds(base, C)], st_v.at[p], sem_in)
            pltpu.async_copy(idx1_h.at[pl.ds(base * NSIG, G)],
                             i1_v.at[p, pl.ds(0, G)], sem_in)
            pltpu.async_copy(idx2_h.at[pl.ds(base * NSIG, G)],
                             i2_v.at[p, pl.ds(0, G)], sem_in)
            pltpu.async_copy(idx3_h.at[pl.ds(base * NSIG, G)],
                             i3_v.at[p, pl.ds(0, G)], sem_in)

        def wait_in(p):
            pltpu.make_async_copy(states_h.at[pl.ds(0, C)], st_v.at[p], sem_in).wait()
            pltpu.make_async_copy(idx1_h.at[pl.ds(0, G)],
                                  i1_v.at[p, pl.ds(0, G)], sem_in).wait()
            pltpu.make_async_copy(idx2_h.at[pl.ds(0, G)],
                                  i2_v.at[p, pl.ds(0, G)], sem_in).wait()
            pltpu.make_async_copy(idx3_h.at[pl.ds(0, G)],
                                  i3_v.at[p, pl.ds(0, G)], sem_in).wait()

        def wait_out(p):
            for o_v in (o1_v, o2_v, o3_v):
                pltpu.make_async_copy(
                    o_v.at[p], o1_h.at[pl.ds(0, C)], sem_out[p]).wait()

        fire_in(0, 0)

        def outer_body(o, _):
            for p in range(2):
                t = o * 2 + p

                @pl.when(t + 1 < n_chunks)
                def _():
                    fire_in(t + 1, (p + 1) % 2)

                wait_in(p)

                @pl.when(t >= 2)
                def _():
                    wait_out(p)

                def pos_body(j, _):
                    r = j * NSIG
                    iv1 = i1_v[p, pl.ds(r, NLANE)]
                    iv2 = i2_v[p, pl.ds(r, NLANE)]
                    iv3 = i3_v[p, pl.ds(r, NLANE)]
                    a, b, c0 = iv1[0], iv1[1], iv1[2]
                    d0, e, f = iv2[0], iv2[1], iv2[2]
                    g, h, i = iv3[0], iv3[1], iv3[2]
                    for c in range(NCOL):
                        sl = pl.ds(c * NLANE, NLANE)
                        acc = st_v[p, j, sl]
                        acc = acc + e1_v[a, sl] + e1_v[b, sl] + e1_v[c0, sl]
                        o1_v[p, j, sl] = acc
                        acc = acc + e2_v[d0, sl] + e2_v[e, sl] + e2_v[f, sl]
                        o2_v[p, j, sl] = acc
                        acc = acc + e3_v[g, sl] + e3_v[h, sl] + e3_v[i, sl]
                        o3_v[p, j, sl] = acc
                    return 0

                lax.fori_loop(0, C, pos_body, 0)

                base = w_base + t * C
                pltpu.async_copy(o1_v.at[p], o1_h.at[pl.ds(base, C)], sem_out[p])
                pltpu.async_copy(o2_v.at[p], o2_h.at[pl.ds(base, C)], sem_out[p])
                pltpu.async_copy(o3_v.at[p], o3_h.at[pl.ds(base, C)], sem_out[p])
            return 0

        lax.fori_loop(0, n_chunks // 2, outer_body, 0)
        wait_out(0)
        wait_out(1)

    return sc_kernel


def kernel(states, signals, emb1, emb2, emb3):
    b, t, d = states.shape
    n_pos = b * t
    st_flat = states.reshape(n_pos, d)
    sig = signals.astype(jnp.int32)
    idx1 = sig[:, 1::S, :].reshape(-1)
    idx2 = sig[:, 2::S, :].reshape(-1)
    idx3 = sig[:, 3::S, :].reshape(-1)
    o1, o2, o3 = _build_kernel(n_pos)(
        st_flat, idx1, idx2, idx3, emb1, emb2, emb3)
    shape = states.shape
    return (states, o1.reshape(shape), o2.reshape(shape), o3.reshape(shape))


# speedup vs baseline: 7.1719x; 1.2395x over previous
import functools

import jax
import jax.numpy as jnp
from jax import lax
from jax.experimental import pallas as pl
from jax.experimental.pallas import tpu as pltpu
from jax.experimental.pallas import tpu_sc as plsc

S = 4
D = 128
NSIG = 3
NLANE = 16
NCOL = D // NLANE

NC = 2
NS = 16
NW = NC * NS

C = 16
G = C * NSIG
LEVELS = 256


def _build_kernel(n_pos):
    per_w = n_pos // NW
    n_chunks = per_w // C
    assert n_chunks % 2 == 0
    mesh = plsc.VectorSubcoreMesh(core_axis_name="c", subcore_axis_name="s")
    f32 = jnp.float32
    i32 = jnp.int32

    @functools.partial(
        pl.kernel,
        mesh=mesh,
        out_type=[jax.ShapeDtypeStruct((n_pos, D), f32)] * 3,
        scratch_types=[
            pltpu.VMEM((LEVELS, D), f32),
            pltpu.VMEM((LEVELS, D), f32),
            pltpu.VMEM((LEVELS, D), f32),
            pltpu.VMEM((2, C, D), f32),
            pltpu.VMEM((2, G + NLANE), i32),
            pltpu.VMEM((2, G + NLANE), i32),
            pltpu.VMEM((2, G + NLANE), i32),
            pltpu.VMEM((2, C, D), f32),
            pltpu.VMEM((2, C, D), f32),
            pltpu.VMEM((2, C, D), f32),
            pltpu.SemaphoreType.DMA,
            pltpu.SemaphoreType.DMA,
            pltpu.SemaphoreType.DMA,
        ],
    )
    def sc_kernel(states_h, idx1_h, idx2_h, idx3_h, e1_h, e2_h, e3_h,
                  o1_h, o2_h, o3_h,
                  e1_v, e2_v, e3_v, st_v, i1_v, i2_v, i3_v,
                  o1_v, o2_v, o3_v, sem_in, sem_o0, sem_o1):
        wid = lax.axis_index("s") * NC + lax.axis_index("c")
        w_base = wid * per_w
        sem_out = (sem_o0, sem_o1)

        pltpu.sync_copy(e1_h, e1_v)
        pltpu.sync_copy(e2_h, e2_v)
        pltpu.sync_copy(e3_h, e3_v)

        def fire_in(t, p):
            base = w_base + t * C
            pltpu.async_copy(states_h.at[pl.ds(base, C)], st_v.at[p], sem_in)
            pltpu.async_copy(idx1_h.at[pl.ds(base * NSIG, G)],
                             i1_v.at[p, pl.ds(0, G)], sem_in)
            pltpu.async_copy(idx2_h.at[pl.ds(base * NSIG, G)],
                             i2_v.at[p, pl.ds(0, G)], sem_in)
            pltpu.async_copy(idx3_h.at[pl.ds(base * NSIG, G)],
                             i3_v.at[p, pl.ds(0, G)], sem_in)

        def wait_in(p):
            pltpu.make_async_copy(states_h.at[pl.ds(0, C)], st_v.at[p], sem_in).wait()
            pltpu.make_async_copy(idx1_h.at[pl.ds(0, G)],
                                  i1_v.at[p, pl.ds(0, G)], sem_in).wait()
            pltpu.make_async_copy(idx2_h.at[pl.ds(0, G)],
                                  i2_v.at[p, pl.ds(0, G)], sem_in).wait()
            pltpu.make_async_copy(idx3_h.at[pl.ds(0, G)],
                                  i3_v.at[p, pl.ds(0, G)], sem_in).wait()

        def wait_out(p):
            for o_v in (o1_v, o2_v, o3_v):
                pltpu.make_async_copy(
                    o_v.at[p], o1_h.at[pl.ds(0, C)], sem_out[p]).wait()

        fire_in(0, 0)

        def outer_body(o, _):
            for p in range(2):
                t = o * 2 + p

                @pl.when(t + 1 < n_chunks)
                def _():
                    fire_in(t + 1, (p + 1) % 2)

                wait_in(p)

                @pl.when(t >= 2)
                def _():
                    wait_out(p)

                @plsc.parallel_loop(0, C, unroll=4)
                def pos_body(j):
                    r = j * NSIG
                    iv1 = i1_v[p, pl.ds(r, NLANE)]
                    iv2 = i2_v[p, pl.ds(r, NLANE)]
                    iv3 = i3_v[p, pl.ds(r, NLANE)]
                    a, b, c0 = iv1[0], iv1[1], iv1[2]
                    d0, e, f = iv2[0], iv2[1], iv2[2]
                    g, h, i = iv3[0], iv3[1], iv3[2]
                    for c in range(NCOL):
                        sl = pl.ds(c * NLANE, NLANE)
                        acc = st_v[p, j, sl]
                        acc = acc + e1_v[a, sl] + e1_v[b, sl] + e1_v[c0, sl]
                        o1_v[p, j, sl] = acc
                        acc = acc + e2_v[d0, sl] + e2_v[e, sl] + e2_v[f, sl]
                        o2_v[p, j, sl] = acc
                        acc = acc + e3_v[g, sl] + e3_v[h, sl] + e3_v[i, sl]
                        o3_v[p, j, sl] = acc

                base = w_base + t * C
                pltpu.async_copy(o1_v.at[p], o1_h.at[pl.ds(base, C)], sem_out[p])
                pltpu.async_copy(o2_v.at[p], o2_h.at[pl.ds(base, C)], sem_out[p])
                pltpu.async_copy(o3_v.at[p], o3_h.at[pl.ds(base, C)], sem_out[p])
            return 0

        lax.fori_loop(0, n_chunks // 2, outer_body, 0)
        wait_out(0)
        wait_out(1)

    return sc_kernel


def kernel(states, signals, emb1, emb2, emb3):
    b, t, d = states.shape
    n_pos = b * t
    st_flat = states.reshape(n_pos, d)
    sig = signals.astype(jnp.int32)
    idx1 = sig[:, 1::S, :].reshape(-1)
    idx2 = sig[:, 2::S, :].reshape(-1)
    idx3 = sig[:, 3::S, :].reshape(-1)
    o1, o2, o3 = _build_kernel(n_pos)(
        st_flat, idx1, idx2, idx3, emb1, emb2, emb3)
    shape = states.shape
    return (states, o1.reshape(shape), o2.reshape(shape), o3.reshape(shape))


# speedup vs baseline: 8.8017x; 1.2272x over previous
import functools

import jax
import jax.numpy as jnp
from jax import lax
from jax.experimental import pallas as pl
from jax.experimental.pallas import tpu as pltpu
from jax.experimental.pallas import tpu_sc as plsc

S = 4
D = 128
NSIG = 3
NLANE = 16
NCOL = D // NLANE
W = S * NSIG

NC = 2
NS = 16
NW = NC * NS

CH = 8
LEVELS = 256


def _build_kernel(b_dim, t_dim):
    n_chunks = t_dim // CH
    assert b_dim == 2 * NW and n_chunks % 4 == 0
    mesh = plsc.VectorSubcoreMesh(core_axis_name="c", subcore_axis_name="s")
    f32 = jnp.float32
    i32 = jnp.int32
    out_sds = jax.ShapeDtypeStruct((b_dim, t_dim, D), f32)
    SGW = CH * W
    SGB = SGW + NLANE
    buf = pltpu.VMEM((2, CH, D), f32)

    @functools.partial(
        pl.kernel,
        mesh=mesh,
        out_type=[out_sds] * 4,
        scratch_types=[
            pltpu.VMEM((3 * LEVELS, D), f32),
            buf, buf, buf, buf,
            pltpu.VMEM((SGB,), i32),
            pltpu.VMEM((SGB,), i32),
            pltpu.VMEM((SGB,), i32),
            pltpu.VMEM((SGB,), i32),
            buf, buf,
            buf, buf,
            buf, buf,
            pltpu.SemaphoreType.DMA,
            pltpu.SemaphoreType.DMA,
            pltpu.SemaphoreType.DMA,
        ],
    )
    def sc_kernel(states_h, sig_h, e1_h, e2_h, e3_h,
                  o0_h, o1_h, o2_h, o3_h,
                  et_v, st0, st1, st2, st3,
                  sg00, sg01, sg10, sg11,
                  o1a, o1b, o2a, o2b, o3a, o3b,
                  sem_in, sem_o0, sem_o1):
        wid = lax.axis_index("s") * NC + lax.axis_index("c")
        b0 = wid * 2
        sem_out = (sem_o0, sem_o1)
        st_l = (st0, st1, st2, st3)
        sg_l = ((sg00, sg01), (sg10, sg11))
        o1_l, o2_l, o3_l = (o1a, o1b), (o2a, o2b), (o3a, o3b)

        pltpu.sync_copy(e1_h, et_v.at[pl.ds(0, LEVELS)])
        pltpu.sync_copy(e2_h, et_v.at[pl.ds(LEVELS, LEVELS)])
        pltpu.sync_copy(e3_h, et_v.at[pl.ds(2 * LEVELS, LEVELS)])

        def fire_in(t, p4, p2):
            ti = t * CH
            pltpu.async_copy(states_h.at[pl.ds(b0, 2), pl.ds(ti, CH)],
                             st_l[p4], sem_in)
            for b2 in range(2):
                off = ((b0 + b2) * t_dim + ti) * W
                pltpu.async_copy(sig_h.at[pl.ds(off, SGB)],
                                 sg_l[p2][b2], sem_in)

        def wait_in(p4, p2):
            pltpu.make_async_copy(states_h.at[pl.ds(0, 2), pl.ds(0, CH)],
                                  st_l[p4], sem_in).wait()
            for b2 in range(2):
                pltpu.make_async_copy(sig_h.at[pl.ds(0, SGB)],
                                      sg_l[p2][b2], sem_in).wait()

        def wait_out(p2):
            for _ in range(4):
                pltpu.make_async_copy(
                    o1_l[p2], o1_h.at[pl.ds(0, 2), pl.ds(0, CH)],
                    sem_out[p2]).wait()

        fire_in(0, 0, 0)

        def outer_body(o, _):
            for p in range(4):
                t = o * 4 + p
                p2 = p % 2

                wait_in(p, p2)

                @pl.when(t >= 2)
                def _():
                    wait_out(p2)

                @pl.when(t + 1 < n_chunks)
                def _():
                    fire_in(t + 1, (p + 1) % 4, (p + 1) % 2)

                for b2 in range(2):
                    sg = sg_l[p2][b2]

                    @plsc.parallel_loop(0, CH, unroll=4)
                    def pos_body(j):
                        iv = sg[pl.ds(j * W, NLANE)]
                        a, b, c0 = iv[3], iv[4], iv[5]
                        d0, e, f = (iv[6] + LEVELS, iv[7] + LEVELS,
                                    iv[8] + LEVELS)
                        g, h, i = (iv[9] + 2 * LEVELS, iv[10] + 2 * LEVELS,
                                   iv[11] + 2 * LEVELS)
                        for c in range(NCOL):
                            sl = pl.ds(c * NLANE, NLANE)
                            acc = st_l[p][b2, j, sl]
                            acc = acc + et_v[a, sl] + et_v[b, sl] + et_v[c0, sl]
                            o1_l[p2][b2, j, sl] = acc
                            acc = acc + et_v[d0, sl] + et_v[e, sl] + et_v[f, sl]
                            o2_l[p2][b2, j, sl] = acc
                            acc = acc + et_v[g, sl] + et_v[h, sl] + et_v[i, sl]
                            o3_l[p2][b2, j, sl] = acc

                ti = t * CH
                db, dt = pl.ds(b0, 2), pl.ds(ti, CH)
                pltpu.async_copy(st_l[p], o0_h.at[db, dt], sem_out[p2])
                pltpu.async_copy(o1_l[p2], o1_h.at[db, dt], sem_out[p2])
                pltpu.async_copy(o2_l[p2], o2_h.at[db, dt], sem_out[p2])
                pltpu.async_copy(o3_l[p2], o3_h.at[db, dt], sem_out[p2])
            return 0

        lax.fori_loop(0, n_chunks // 4, outer_body, 0)
        wait_out(0)
        wait_out(1)

    return sc_kernel


def kernel(states, signals, emb1, emb2, emb3):
    b_dim, t_dim, _ = states.shape
    sig_flat = signals.astype(jnp.int32).reshape(-1)
    sig_flat = jnp.concatenate([sig_flat, jnp.zeros((NLANE,), jnp.int32)])
    return tuple(_build_kernel(b_dim, t_dim)(states, sig_flat, emb1, emb2, emb3))
